# baseline (speedup 1.0000x reference)
import jax
import jax.numpy as jnp
from jax.experimental import pallas as pl
from jax.experimental.pallas import tpu as pltpu

N_EDGES = 320000
DIM_EMB = 128
DIM_DICT_ROWS = 400

_R = 4000
_B = N_EDGES // _R


def _tc_body(idx_ref, tab_ref, out_ref):
    idx2 = idx_ref[0]
    iota = jax.lax.broadcasted_iota(jnp.int32, (_R, DIM_DICT_ROWS), 1)
    oh = (idx2 == iota).astype(jnp.bfloat16)
    out_ref[...] = jax.lax.dot_general(
        oh, tab_ref[...], (((1,), (0,)), ((), ())),
        preferred_element_type=jnp.float32)


def kernel(edge_type, embedding):
    idx3 = edge_type.astype(jnp.int32).reshape(_B, _R, 1)
    tab_bf = embedding.astype(jnp.bfloat16)
    out = pl.pallas_call(
        _tc_body,
        grid=(_B,),
        in_specs=[
            pl.BlockSpec((1, _R, 1), lambda i: (i, 0, 0)),
            pl.BlockSpec((DIM_DICT_ROWS, DIM_EMB), lambda i: (0, 0)),
        ],
        out_specs=pl.BlockSpec((_R, DIM_EMB), lambda i: (i, 0)),
        out_shape=jax.ShapeDtypeStruct((N_EDGES, DIM_EMB), jnp.float32),
    )(idx3, tab_bf)
    return out


# speedup vs baseline: 3.6916x; 3.6916x over previous
import functools

import jax
import jax.numpy as jnp
from jax import lax
from jax.experimental import pallas as pl
from jax.experimental.pallas import tpu as pltpu, tpu_sc as plsc

N_EDGES = 320000
DIM_EMB = 128
DIM_DICT_ROWS = 400

_CHUNK = 80
_NCHUNK = N_EDGES // _CHUNK
_NB = 10
_LA = 5


def _make_kernel(n_workers: int):
    cpw = _NCHUNK // n_workers
    mesh = plsc.VectorSubcoreMesh(core_axis_name="c", subcore_axis_name="s")

    @functools.partial(
        pl.kernel,
        mesh=mesh,
        out_type=jax.ShapeDtypeStruct((_NCHUNK, _CHUNK, DIM_EMB), jnp.float32),
        scratch_types=[
            pltpu.VMEM((cpw, _CHUNK), jnp.int32),
            pltpu.VMEM_SHARED((DIM_DICT_ROWS, DIM_EMB), jnp.float32),
            pltpu.VMEM((_NB, _CHUNK, DIM_EMB), jnp.float32),
            *([pltpu.SemaphoreType.DMA] * _NB),
            *([pltpu.SemaphoreType.DMA] * _NB),
        ],
    )
    def k(et_hbm, table_hbm, out_hbm, idx_all, table_sh, rows, *sems):
        gsems, wsems = sems[:_NB], sems[_NB:]
        wid = lax.axis_index("s") * 2 + lax.axis_index("c")
        base = wid * cpw

        pltpu.sync_copy(et_hbm.at[wid], idx_all)

        @pl.when(lax.axis_index("s") == 0)
        def _stage_table():
            pltpu.sync_copy(table_hbm, table_sh)

        plsc.subcore_barrier()

        def gather(c, b):
            pltpu.async_copy(table_sh.at[idx_all.at[c]], rows.at[b], gsems[b])

        def wait_gather(c, b):
            pltpu.make_async_copy(
                table_sh.at[idx_all.at[c]], rows.at[b], gsems[b]).wait()

        def write(c, b):
            pltpu.async_copy(rows.at[b], out_hbm.at[base + c], wsems[b])

        def wait_write(c, b):
            pltpu.make_async_copy(
                rows.at[b], out_hbm.at[base + c], wsems[b]).wait()

        for c in range(_LA):
            gather(c, c % _NB)
        for c in range(_LA):
            wait_gather(c, c % _NB)
            write(c, c % _NB)
            gather(c + _LA, (c + _LA) % _NB)

        def step(c, k_):
            b = (_LA + k_) % _NB
            wait_gather(c, b)
            write(c, b)
            wait_write(c - _LA, (b + _LA) % _NB)
            gather(c + _LA, (b + _LA) % _NB)

        @pl.loop(0, (cpw - 2 * _LA) // _NB)
        def grp(g):
            for k_ in range(_NB):
                step(_LA + g * _NB + k_, k_)

        main_end = _LA + ((cpw - 2 * _LA) // _NB) * _NB
        for c in range(main_end, cpw - _LA):
            step(c, c - _LA)
        for c in range(cpw - _LA, cpw):
            b = c % _NB
            wait_gather(c, b)
            write(c, b)
            wait_write(c - _LA, (b + _LA) % _NB)
        for c in range(cpw - _LA, cpw):
            wait_write(c, c % _NB)

    return k


def kernel(edge_type, embedding):
    et = edge_type.astype(jnp.int32).reshape(32, _NCHUNK // 32, _CHUNK)
    out = _make_kernel(32)(et, embedding)
    return out.reshape(N_EDGES, DIM_EMB)


# speedup vs baseline: 3.8035x; 1.0303x over previous
import functools

import jax
import jax.numpy as jnp
from jax import lax
from jax.experimental import pallas as pl
from jax.experimental.pallas import tpu as pltpu, tpu_sc as plsc

N_EDGES = 320000
DIM_EMB = 128
DIM_DICT_ROWS = 400

_CHUNK = 80
_NCHUNK = N_EDGES // _CHUNK
_NB = 10
_LA = 5
_TS = 40


def _make_kernel(n_workers: int):
    cpw = _NCHUNK // n_workers
    ids_pw = cpw * _CHUNK
    mesh = plsc.VectorSubcoreMesh(core_axis_name="c", subcore_axis_name="s")

    @functools.partial(
        pl.kernel,
        mesh=mesh,
        out_type=jax.ShapeDtypeStruct((_NCHUNK, _CHUNK, DIM_EMB), jnp.float32),
        scratch_types=[
            pltpu.VMEM((ids_pw,), jnp.int32),
            pltpu.VMEM_SHARED((DIM_DICT_ROWS, DIM_EMB), jnp.float32),
            pltpu.VMEM((_NB, _CHUNK, DIM_EMB), jnp.float32),
            pltpu.SemaphoreType.DMA,
            *([pltpu.SemaphoreType.DMA] * _NB),
            *([pltpu.SemaphoreType.DMA] * _NB),
        ],
    )
    def k(et_hbm, table_hbm, out_hbm, idx_all, table_sh, rows, ssem, *sems):
        gsems, wsems = sems[:_NB], sems[_NB:]
        sid = lax.axis_index("s")
        wid = sid * 2 + lax.axis_index("c")
        base = wid * cpw

        ids_cp = pltpu.async_copy(
            et_hbm.at[pl.ds(pl.multiple_of(wid * ids_pw, 8), ids_pw)],
            idx_all, ssem)

        @pl.when(sid < DIM_DICT_ROWS // _TS)
        def _stage_table():
            pltpu.async_copy(
                table_hbm.at[pl.ds(pl.multiple_of(sid * _TS, 8), _TS)],
                table_sh.at[pl.ds(pl.multiple_of(sid * _TS, 8), _TS)],
                ssem).wait()

        ids_cp.wait()
        plsc.subcore_barrier()

        def idx_at(c):
            return idx_all.at[pl.ds(pl.multiple_of(c * _CHUNK, 8), _CHUNK)]

        def gather(c, b):
            pltpu.async_copy(table_sh.at[idx_at(c)], rows.at[b], gsems[b])

        def wait_gather(c, b):
            pltpu.make_async_copy(
                table_sh.at[idx_at(c)], rows.at[b], gsems[b]).wait()

        def write(c, b):
            pltpu.async_copy(rows.at[b], out_hbm.at[base + c], wsems[b])

        def wait_write(c, b):
            pltpu.make_async_copy(
                rows.at[b], out_hbm.at[base + c], wsems[b]).wait()

        for c in range(_LA):
            gather(c, c % _NB)
        for c in range(_LA):
            wait_gather(c, c % _NB)
            write(c, c % _NB)
            gather(c + _LA, (c + _LA) % _NB)

        def step(c, k_):
            b = (_LA + k_) % _NB
            wait_gather(c, b)
            write(c, b)
            wait_write(c - _LA, (b + _LA) % _NB)
            gather(c + _LA, (b + _LA) % _NB)

        @pl.loop(0, (cpw - 2 * _LA) // _NB)
        def grp(g):
            for k_ in range(_NB):
                step(_LA + g * _NB + k_, k_)

        main_end = _LA + ((cpw - 2 * _LA) // _NB) * _NB
        for c in range(main_end, cpw - _LA):
            step(c, c - _LA)
        for c in range(cpw - _LA, cpw):
            b = c % _NB
            wait_gather(c, b)
            write(c, b)
            wait_write(c - _LA, (b + _LA) % _NB)
        for c in range(cpw - _LA, cpw):
            wait_write(c, c % _NB)

    return k


def kernel(edge_type, embedding):
    et = edge_type.astype(jnp.int32)
    out = _make_kernel(32)(et, embedding)
    return out.reshape(N_EDGES, DIM_EMB)
